# baseline (speedup 1.0000x reference)
import functools

import jax
import jax.numpy as jnp
from jax import lax
from jax.experimental import pallas as pl
from jax.experimental.pallas import tpu as pltpu

T = 2048
H = 1024
FF = 4096
E = 8
NE = 10
TOPK = 2
TM = 256
NB = (T * TOPK) // TM + E
NPAD = NB * TM
FFC = 512
NFF = FF // FFC

_F32 = jnp.float32
_BF16 = jnp.bfloat16


def _router_body(x_ref, g_ref, logits_ref, meta_ref):
    x = x_ref[...]
    g = g_ref[...]
    logits = lax.dot_general(x, g, (((1,), (1,)), ((), ())),
                             preferred_element_type=_F32)
    logits_ref[...] = logits
    lane = lax.broadcasted_iota(jnp.int32, (T, 16), 1)
    masked = jnp.where(lane < NE, logits, -1e30)
    m = jnp.max(masked, axis=1, keepdims=True)
    ex = jnp.exp(masked - m)
    p = ex / jnp.sum(ex, axis=1, keepdims=True)
    p1 = jnp.max(p, axis=1, keepdims=True)
    i1 = jnp.min(jnp.where(p == p1, lane, 999), axis=1, keepdims=True)
    pm = jnp.where(lane == i1, -1.0, p)
    p2 = jnp.max(pm, axis=1, keepdims=True)
    i2 = jnp.min(jnp.where(pm == p2, lane, 999), axis=1, keepdims=True)
    m1 = (i1 < E).astype(_F32)
    m2 = (i2 < E).astype(_F32)
    s = p1 * m1 + p2 * m2
    d = jnp.where(s == 0.0, 1.0, s)
    w1 = p1 * m1 / d
    w2 = p2 * m2 / d
    lane8 = lax.broadcasted_iota(jnp.int32, (T, 8), 1)
    meta = jnp.where(lane8 == 0, w1,
                     jnp.where(lane8 == 1, w2,
                               jnp.where(lane8 == 2, i1.astype(_F32),
                                         i2.astype(_F32))))
    meta_ref[...] = meta


def _moe_body(be_ref, na_ref, tok_ref, ww_ref, xb_ref,
              w1_ref, w3_ref, w2_ref, out_ref, gb_scr, xt_scr, oacc_scr):
    b = pl.program_id(0)
    f = pl.program_id(1)
    active = b < na_ref[0]

    @pl.when((b == 0) & (f == 0))
    def _init():
        out_ref[...] = jnp.zeros_like(out_ref)

    @pl.when(active & (f == 0))
    def _gather():
        ids = tok_ref[0, 0, :]
        cols = lax.broadcasted_iota(jnp.int32, (TM, T), 1)
        gb = (ids[:, None] == cols).astype(_BF16)
        gb_scr[...] = gb
        xt_scr[...] = jnp.dot(gb, xb_ref[...],
                              preferred_element_type=_F32).astype(_BF16)
        oacc_scr[...] = jnp.zeros_like(oacc_scr)

    @pl.when(active)
    def _ffn():
        xt = xt_scr[...]
        a = lax.dot_general(xt, w1_ref[0], (((1,), (1,)), ((), ())),
                            preferred_element_type=_F32)
        c = lax.dot_general(xt, w3_ref[0], (((1,), (1,)), ((), ())),
                            preferred_element_type=_F32)
        hh = ((a * jax.nn.sigmoid(a)) * c).astype(_BF16)
        oacc_scr[...] += lax.dot_general(hh, w2_ref[0],
                                         (((1,), (1,)), ((), ())),
                                         preferred_element_type=_F32)

    @pl.when(active & (f == NFF - 1))
    def _scatter():
        ww = ww_ref[0, 0, :]
        ow = (oacc_scr[...] * ww[:, None]).astype(_BF16)
        gb = gb_scr[...]
        out_ref[...] += lax.dot_general(gb, ow, (((0,), (0,)), ((), ())),
                                        preferred_element_type=_F32)


def kernel(hidden_states, gate_w, gate2_w, W1, W2, W3):
    b, s, h = hidden_states.shape
    x = hidden_states.reshape(T, H)

    gates = jnp.concatenate(
        [gate_w, gate2_w, jnp.zeros((16 - NE, H), _F32)], axis=0)

    logits16, meta = pl.pallas_call(
        _router_body,
        out_shape=(jax.ShapeDtypeStruct((T, 16), _F32),
                   jax.ShapeDtypeStruct((T, 8), _F32)),
    )(x, gates)

    router_logits = logits16[:, :NE]

    wts = meta[:, 0:2]
    eids = meta[:, 2:4].astype(jnp.int32)
    e_flat = eids.reshape(-1)
    w_flat = wts.reshape(-1)
    tok = jnp.arange(T * TOPK, dtype=jnp.int32) // TOPK
    key = jnp.where(e_flat < E, e_flat, E)
    cnt = jnp.bincount(key, length=E + 1)[:E].astype(jnp.int32)
    blocks_per = (cnt + TM - 1) // TM
    blk_start = jnp.concatenate(
        [jnp.zeros((1,), jnp.int32), jnp.cumsum(blocks_per)[:-1]])
    nact = jnp.sum(blocks_per).astype(jnp.int32)
    run_start = jnp.concatenate(
        [jnp.zeros((1,), jnp.int32), jnp.cumsum(cnt)])
    perm = jnp.argsort(key, stable=True)
    se = key[perm]
    st = tok[perm]
    sw = w_flat[perm]
    pos_in_run = jnp.arange(T * TOPK, dtype=jnp.int32) - run_start[se]
    dest = jnp.where(se < E,
                     TM * blk_start[jnp.minimum(se, E - 1)] + pos_in_run,
                     NPAD)
    row_tok = jnp.zeros((NPAD + 1,), jnp.int32).at[dest].set(st)[:NPAD]
    row_w = jnp.zeros((NPAD + 1,), _F32).at[dest].set(sw)[:NPAD]
    bidx = jnp.arange(NB, dtype=jnp.int32)
    be = jnp.sum(bidx[:, None] >= blk_start[None, :], axis=1).astype(jnp.int32) - 1
    be_last = be[jnp.maximum(nact - 1, 0)]
    be = jnp.where(bidx < nact, be, be_last)
    nact_arr = nact.reshape(1)

    xb = x.astype(_BF16)
    w1b = W1.astype(_BF16)
    w3b = W3.astype(_BF16)
    w2b = W2.astype(_BF16)
    tok3 = row_tok.reshape(NB, 1, TM)
    ww3 = row_w.reshape(NB, 1, TM)

    grid_spec = pltpu.PrefetchScalarGridSpec(
        num_scalar_prefetch=2,
        grid=(NB, NFF),
        in_specs=[
            pl.BlockSpec((1, 1, TM), lambda b, f, be, na: (b, 0, 0)),
            pl.BlockSpec((1, 1, TM), lambda b, f, be, na: (b, 0, 0)),
            pl.BlockSpec((T, H), lambda b, f, be, na: (0, 0)),
            pl.BlockSpec((1, FFC, H), lambda b, f, be, na: (be[b], f, 0)),
            pl.BlockSpec((1, FFC, H), lambda b, f, be, na: (be[b], f, 0)),
            pl.BlockSpec((1, H, FFC), lambda b, f, be, na: (be[b], 0, f)),
        ],
        out_specs=pl.BlockSpec((T, H), lambda b, f, be, na: (0, 0)),
        scratch_shapes=[
            pltpu.VMEM((TM, T), _BF16),
            pltpu.VMEM((TM, H), _BF16),
            pltpu.VMEM((TM, H), _F32),
        ],
    )

    final = pl.pallas_call(
        _moe_body,
        grid_spec=grid_spec,
        out_shape=jax.ShapeDtypeStruct((T, H), _F32),
        compiler_params=pltpu.CompilerParams(
            dimension_semantics=("arbitrary", "arbitrary")),
    )(be, nact_arr, tok3, ww3, xb, w1b, w3b, w2b)

    return final.reshape(b, s, h), router_logits


# speedup vs baseline: 1.3077x; 1.3077x over previous
import functools

import jax
import jax.numpy as jnp
from jax import lax
from jax.experimental import pallas as pl
from jax.experimental.pallas import tpu as pltpu

T = 2048
H = 1024
FF = 4096
E = 8
NE = 10
TOPK = 2
TM = 256
NB = (T * TOPK) // TM + E
NPAD = NB * TM
FFC = 512
NFF = FF // FFC

_F32 = jnp.float32
_BF16 = jnp.bfloat16


def _router_body(x_ref, g_ref, logits_ref, meta_ref):
    x = x_ref[...]
    g = g_ref[...]
    logits = lax.dot_general(x, g, (((1,), (1,)), ((), ())),
                             preferred_element_type=_F32)
    logits_ref[...] = logits
    lane = lax.broadcasted_iota(jnp.int32, (T, 16), 1)
    masked = jnp.where(lane < NE, logits, -1e30)
    m = jnp.max(masked, axis=1, keepdims=True)
    ex = jnp.exp(masked - m)
    p = ex / jnp.sum(ex, axis=1, keepdims=True)
    p1 = jnp.max(p, axis=1, keepdims=True)
    i1 = jnp.min(jnp.where(p == p1, lane, 999), axis=1, keepdims=True)
    pm = jnp.where(lane == i1, -1.0, p)
    p2 = jnp.max(pm, axis=1, keepdims=True)
    i2 = jnp.min(jnp.where(pm == p2, lane, 999), axis=1, keepdims=True)
    m1 = (i1 < E).astype(_F32)
    m2 = (i2 < E).astype(_F32)
    s = p1 * m1 + p2 * m2
    d = jnp.where(s == 0.0, 1.0, s)
    w1 = p1 * m1 / d
    w2 = p2 * m2 / d
    lane8 = lax.broadcasted_iota(jnp.int32, (T, 8), 1)
    meta = jnp.where(lane8 == 0, w1,
                     jnp.where(lane8 == 1, w2,
                               jnp.where(lane8 == 2, i1.astype(_F32),
                                         i2.astype(_F32))))
    meta_ref[...] = meta


def _moe_body(be_ref, na_ref, tok_ref, ww_ref, xb_ref,
              w1_ref, w3_ref, w2_ref, out_ref, gb_scr):
    b = pl.program_id(0)
    active = b < na_ref[0]

    @pl.when(b == 0)
    def _init():
        out_ref[...] = jnp.zeros_like(out_ref)

    @pl.when(active)
    def _compute():
        ids = tok_ref[0, 0, :]
        cols = lax.broadcasted_iota(jnp.int32, (TM, T), 1)
        gb = (ids[:, None] == cols).astype(_BF16)
        gb_scr[...] = gb
        xt = jnp.dot(gb, xb_ref[...],
                     preferred_element_type=_F32).astype(_BF16)
        oacc = jnp.zeros((TM, H), _F32)
        for fi in range(NFF):
            w1c = w1_ref[0, fi * FFC:(fi + 1) * FFC, :]
            w3c = w3_ref[0, fi * FFC:(fi + 1) * FFC, :]
            w2c = w2_ref[0, :, fi * FFC:(fi + 1) * FFC]
            a = lax.dot_general(xt, w1c, (((1,), (1,)), ((), ())),
                                preferred_element_type=_F32)
            c = lax.dot_general(xt, w3c, (((1,), (1,)), ((), ())),
                                preferred_element_type=_F32)
            hh = ((a * jax.nn.sigmoid(a)) * c).astype(_BF16)
            oacc = oacc + lax.dot_general(hh, w2c, (((1,), (1,)), ((), ())),
                                          preferred_element_type=_F32)
        ww = ww_ref[0, 0, :]
        ow = (oacc * ww[:, None]).astype(_BF16)
        out_ref[...] += lax.dot_general(gb_scr[...], ow,
                                        (((0,), (0,)), ((), ())),
                                        preferred_element_type=_F32)


def kernel(hidden_states, gate_w, gate2_w, W1, W2, W3):
    b, s, h = hidden_states.shape
    x = hidden_states.reshape(T, H)

    gates = jnp.concatenate(
        [gate_w, gate2_w, jnp.zeros((16 - NE, H), _F32)], axis=0)

    logits16, meta = pl.pallas_call(
        _router_body,
        out_shape=(jax.ShapeDtypeStruct((T, 16), _F32),
                   jax.ShapeDtypeStruct((T, 8), _F32)),
    )(x, gates)

    router_logits = logits16[:, :NE]

    wts = meta[:, 0:2]
    eids = meta[:, 2:4].astype(jnp.int32)
    e_flat = eids.reshape(-1)
    w_flat = wts.reshape(-1)
    tok = jnp.arange(T * TOPK, dtype=jnp.int32) // TOPK
    key = jnp.where(e_flat < E, e_flat, E)
    cnt = jnp.bincount(key, length=E + 1)[:E].astype(jnp.int32)
    blocks_per = (cnt + TM - 1) // TM
    blk_start = jnp.concatenate(
        [jnp.zeros((1,), jnp.int32), jnp.cumsum(blocks_per)[:-1]])
    nact = jnp.sum(blocks_per).astype(jnp.int32)
    run_start = jnp.concatenate(
        [jnp.zeros((1,), jnp.int32), jnp.cumsum(cnt)])
    perm = jnp.argsort(key, stable=True)
    se = key[perm]
    st = tok[perm]
    sw = w_flat[perm]
    pos_in_run = jnp.arange(T * TOPK, dtype=jnp.int32) - run_start[se]
    dest = jnp.where(se < E,
                     TM * blk_start[jnp.minimum(se, E - 1)] + pos_in_run,
                     NPAD)
    row_tok = jnp.zeros((NPAD + 1,), jnp.int32).at[dest].set(st)[:NPAD]
    row_w = jnp.zeros((NPAD + 1,), _F32).at[dest].set(sw)[:NPAD]
    bidx = jnp.arange(NB, dtype=jnp.int32)
    be = jnp.sum(bidx[:, None] >= blk_start[None, :], axis=1).astype(jnp.int32) - 1
    be_last = be[jnp.maximum(nact - 1, 0)]
    be = jnp.where(bidx < nact, be, be_last)
    nact_arr = nact.reshape(1)

    xb = x.astype(_BF16)
    w1b = W1.astype(_BF16)
    w3b = W3.astype(_BF16)
    w2b = W2.astype(_BF16)
    tok3 = row_tok.reshape(NB, 1, TM)
    ww3 = row_w.reshape(NB, 1, TM)

    grid_spec = pltpu.PrefetchScalarGridSpec(
        num_scalar_prefetch=2,
        grid=(NB,),
        in_specs=[
            pl.BlockSpec((1, 1, TM), lambda b, be, na: (b, 0, 0)),
            pl.BlockSpec((1, 1, TM), lambda b, be, na: (b, 0, 0)),
            pl.BlockSpec((T, H), lambda b, be, na: (0, 0)),
            pl.BlockSpec((1, FF, H), lambda b, be, na: (be[b], 0, 0)),
            pl.BlockSpec((1, FF, H), lambda b, be, na: (be[b], 0, 0)),
            pl.BlockSpec((1, H, FF), lambda b, be, na: (be[b], 0, 0)),
        ],
        out_specs=pl.BlockSpec((T, H), lambda b, be, na: (0, 0)),
        scratch_shapes=[
            pltpu.VMEM((TM, T), _BF16),
        ],
    )

    final = pl.pallas_call(
        _moe_body,
        grid_spec=grid_spec,
        out_shape=jax.ShapeDtypeStruct((T, H), _F32),
        compiler_params=pltpu.CompilerParams(
            dimension_semantics=("arbitrary",),
            vmem_limit_bytes=100 * 1024 * 1024),
    )(be, nact_arr, tok3, ww3, xb, w1b, w3b, w2b)

    return final.reshape(b, s, h), router_logits
